# baseline (speedup 1.0000x reference)
import functools

import jax
import jax.numpy as jnp
from jax import lax
from jax.experimental import pallas as pl
from jax.experimental.pallas import tpu as pltpu
from jax.experimental.pallas import tpu_sc as plsc

_BATCH = 4096
_SEQ = 200
_DIM = 64
_B = _BATCH * _SEQ

_NC = 2
_NS = 16
_NW = _NC * _NS
_B_PER_W = _B // _NW

_CHUNK = 256
_N_CHUNKS = _B_PER_W // _CHUNK
_N_ROUNDS = _N_CHUNKS // 2


@jax.jit
def _embed(x_flat, table):
  mesh = plsc.VectorSubcoreMesh(core_axis_name="c", subcore_axis_name="s")

  @functools.partial(
      pl.kernel,
      out_type=jax.ShapeDtypeStruct((_B * _DIM,), jnp.float32),
      mesh=mesh,
      scratch_types=[
          pltpu.VMEM((_B_PER_W,), jnp.int32),
          pltpu.VMEM((2, _CHUNK, _DIM), jnp.float32),
          pltpu.SemaphoreType.DMA,
          pltpu.SemaphoreType.DMA,
          pltpu.SemaphoreType.DMA,
          pltpu.SemaphoreType.DMA,
      ],
  )
  def k(x_hbm, table_hbm, out_hbm, idx_v, rows_v, gs0, gs1, ss0, ss1):
    gsems = (gs0, gs1)
    ssems = (ss0, ss1)
    wid = lax.axis_index("s") * _NC + lax.axis_index("c")
    base = wid * _B_PER_W
    pltpu.sync_copy(x_hbm.at[pl.ds(base, _B_PER_W)], idx_v)

    def start_gather(g, b):
      def row16(j, carry):
        vec = idx_v[pl.ds(g * _CHUNK + j * 16, 16)]
        for kk in range(16):
          pltpu.async_copy(
              table_hbm.at[pl.ds(vec[kk], 1)],
              rows_v.at[b].at[pl.ds(j * 16 + kk, 1)],
              gsems[b],
          )
        return carry

      lax.fori_loop(0, _CHUNK // 16, row16, 0)

    def wait_gather(b):
      pltpu.make_async_copy(
          table_hbm.at[pl.ds(0, _CHUNK)], rows_v.at[b], gsems[b]
      ).wait()

    def start_store(g, b):
      def row16(j, carry):
        for kk in range(16):
          r = j * 16 + kk
          pltpu.async_copy(
              rows_v.at[b].at[r],
              out_hbm.at[pl.ds((base + g * _CHUNK + r) * _DIM, _DIM)],
              ssems[b],
          )
        return carry

      lax.fori_loop(0, _CHUNK // 16, row16, 0)

    def wait_store(b):
      pltpu.make_async_copy(
          x_hbm.at[pl.ds(0, _CHUNK * _DIM)],
          idx_v.at[pl.ds(0, _CHUNK * _DIM)],
          ssems[b],
      ).wait()

    start_gather(0, 0)
    start_gather(1, 1)

    def body(o, carry):
      wait_gather(0)
      start_store(2 * o, 0)

      @pl.when(o < _N_ROUNDS - 1)
      def _():
        wait_store(0)
        start_gather(2 * o + 2, 0)

      wait_gather(1)
      start_store(2 * o + 1, 1)

      @pl.when(o < _N_ROUNDS - 1)
      def _():
        wait_store(1)
        start_gather(2 * o + 3, 1)

      return carry

    lax.fori_loop(0, _N_ROUNDS, body, 0)
    wait_store(0)
    wait_store(1)

  return k(x_flat, table)


def kernel(x, table):
  return _embed(x.reshape(_B), table).reshape(_BATCH, _SEQ, _DIM)


# speedup vs baseline: 1.2769x; 1.2769x over previous
import functools

import jax
import jax.numpy as jnp
from jax import lax
from jax.experimental import pallas as pl
from jax.experimental.pallas import tpu as pltpu
from jax.experimental.pallas import tpu_sc as plsc

_BATCH = 4096
_SEQ = 200
_DIM = 64
_B = _BATCH * _SEQ

_NC = 2
_NS = 16
_NW = _NC * _NS
_B_PER_W = _B // _NW

_NBUF = 4
_CHUNK = 160
_N_CHUNKS = _B_PER_W // _CHUNK
_N_ROUNDS = _N_CHUNKS // _NBUF


@jax.jit
def _embed(x_flat, table):
  mesh = plsc.VectorSubcoreMesh(core_axis_name="c", subcore_axis_name="s")

  @functools.partial(
      pl.kernel,
      out_type=jax.ShapeDtypeStruct((_B, _DIM), jnp.float32),
      mesh=mesh,
      scratch_types=[
          pltpu.VMEM((_B_PER_W,), jnp.int32),
          pltpu.VMEM((_NBUF, _CHUNK, _DIM), jnp.float32),
      ]
      + [pltpu.SemaphoreType.DMA] * (2 * _NBUF),
  )
  def k(x_hbm, table_hbm, out_hbm, idx_v, rows_v, *sems):
    gsems = sems[:_NBUF]
    ssems = sems[_NBUF:]
    wid = lax.axis_index("s") * _NC + lax.axis_index("c")
    base = wid * _B_PER_W
    pltpu.sync_copy(x_hbm.at[pl.ds(base, _B_PER_W)], idx_v)

    def start_gather(g, b):
      def row16(j, carry):
        vec = idx_v[pl.ds(g * _CHUNK + j * 16, 16)]
        for kk in range(16):
          pltpu.async_copy(
              table_hbm.at[pl.ds(vec[kk], 1)],
              rows_v.at[b].at[pl.ds(j * 16 + kk, 1)],
              gsems[b],
          )
        return carry

      lax.fori_loop(0, _CHUNK // 16, row16, 0)

    def wait_gather(b):
      pltpu.make_async_copy(
          table_hbm.at[pl.ds(0, _CHUNK)], rows_v.at[b], gsems[b]
      ).wait()

    def start_store(g, b):
      pltpu.async_copy(
          rows_v.at[b], out_hbm.at[pl.ds(base + g * _CHUNK, _CHUNK)], ssems[b]
      )

    def wait_store(b):
      pltpu.make_async_copy(
          rows_v.at[b], out_hbm.at[pl.ds(base, _CHUNK)], ssems[b]
      ).wait()

    for b in range(_NBUF):
      start_gather(b, b)

    def body(o, carry):
      for b in range(_NBUF):
        wait_gather(b)
        start_store(_NBUF * o + b, b)

        @pl.when(o < _N_ROUNDS - 1)
        def _(b=b):
          wait_store(b)
          start_gather(_NBUF * (o + 1) + b, b)

      return carry

    lax.fori_loop(0, _N_ROUNDS, body, 0)
    for b in range(_NBUF):
      wait_store(b)

  return k(x_flat, table)


def kernel(x, table):
  return _embed(x.reshape(_B), table).reshape(_BATCH, _SEQ, _DIM)
